# Initial kernel scaffold; baseline (speedup 1.0000x reference)
import jax
import jax.numpy as jnp
from jax.experimental import pallas as pl


def kernel(uv, feat_2d, feat_3d, w1, b1, w2, b2, wo, bo):
    raise NotImplementedError("write your pallas kernel here")


# speedup vs baseline: 30.8203x; 30.8203x over previous
import functools

import jax
import jax.numpy as jnp
from jax import lax
from jax.experimental import pallas as pl
from jax.experimental.pallas import tpu as pltpu
from jax.experimental.pallas import tpu_sc as plsc

BS, H, W = 2, 96, 320
M = H * W
NP = 8192
KNN = 3
C3 = 64
MB_A = 128
MB_C = 512
DPAD = 128
WIN = 120
NWORK = 32
NSTEP = (BS * KNN * M) // (NWORK * WIN)
NBUF = 4


def _topk_body(uv_ref, xs_ref, ys_ref, idx_ref):
    b = pl.program_id(0)
    px = uv_ref[0, 0:1, :]
    py = uv_ref[0, 1:2, :]
    qx = xs_ref[...]
    qy = ys_ref[...]
    qq = qx * qx + qy * qy
    pp = px * px + py * py
    qxb = qx.astype(jnp.bfloat16).astype(jnp.float32)
    qyb = qy.astype(jnp.bfloat16).astype(jnp.float32)
    pxb = px.astype(jnp.bfloat16).astype(jnp.float32)
    pyb = py.astype(jnp.bfloat16).astype(jnp.float32)
    qp = qxb * pxb + qyb * pyb
    d = (qq - 2.0 * qp) + pp
    iota = lax.broadcasted_iota(jnp.int32, d.shape, 1)
    inf = jnp.float32(jnp.inf)
    big = jnp.int32(NP)

    def sel(dcur):
        m = jnp.min(dcur, axis=1, keepdims=True)
        e = dcur == m
        i = jnp.min(jnp.where(e, iota, big), axis=1, keepdims=True)
        onehot = e & (iota == i)
        return i, onehot

    i1, oh1 = sel(d)
    d2 = jnp.where(oh1, inf, d)
    i2, oh2 = sel(d2)
    d3 = jnp.where(oh2, inf, d2)
    i3, _ = sel(d3)
    idx_ref[0] = jnp.concatenate([i1, i2, i3], axis=1) + b * NP


def _knn_call(uv, xs, ys):
    return pl.pallas_call(
        _topk_body,
        grid=(BS, M // MB_A),
        in_specs=[
            pl.BlockSpec((1, 2, NP), lambda b, m: (b, 0, 0)),
            pl.BlockSpec((MB_A, 1), lambda b, m: (m, 0)),
            pl.BlockSpec((MB_A, 1), lambda b, m: (m, 0)),
        ],
        out_specs=pl.BlockSpec((1, MB_A, KNN), lambda b, m: (b, m, 0)),
        out_shape=jax.ShapeDtypeStruct((BS, M, KNN), jnp.int32),
    )(uv, xs, ys)


def _gather_call(idx2, table):
    mesh = plsc.VectorSubcoreMesh(core_axis_name="c", subcore_axis_name="s")

    @functools.partial(
        pl.kernel,
        out_type=jax.ShapeDtypeStruct((BS * KNN * M, DPAD), jnp.float32),
        mesh=mesh,
        scratch_types=(
            [pltpu.VMEM((NSTEP, WIN), jnp.int32)]
            + [pltpu.VMEM((WIN, DPAD), jnp.float32) for _ in range(NBUF)]
            + [pltpu.SemaphoreType.DMA for _ in range(NBUF)]
        ),
    )
    def gk(idx_hbm, table_hbm, out_hbm, idx_v, rv0, rv1, rv2, rv3,
           sm0, sm1, sm2, sm3):
        bufs = (rv0, rv1, rv2, rv3)
        sems = (sm0, sm1, sm2, sm3)
        cid = lax.axis_index("c")
        sid = lax.axis_index("s")
        wid = sid * 2 + cid
        pltpu.sync_copy(idx_hbm.at[pl.ds(wid * NSTEP, NSTEP)], idx_v)

        def fire(s, t):
            pltpu.async_copy(table_hbm.at[idx_v.at[s]], bufs[t], sems[t])

        def wait_scatter(s, t):
            pltpu.make_async_copy(
                table_hbm.at[idx_v.at[s]], bufs[t], sems[t]).wait()
            pltpu.sync_copy(
                bufs[t], out_hbm.at[pl.ds((wid * NSTEP + s) * WIN, WIN)])

        for t in range(NBUF):
            fire(t, t)

        @pl.loop(0, NSTEP - NBUF, step=NBUF)
        def _(g):
            for t in range(NBUF):
                wait_scatter(g + t, t)
                fire(g + t + NBUF, t)

        for t in range(NBUF):
            wait_scatter(NSTEP - NBUF + t, t)

    return gk(idx2, table)


def _mlp_body(g_ref, xs_ref, ys_ref, w1t_ref, b1_ref, w2_ref, b2_ref,
              wo_ref, bo_ref, out_ref):
    qx = xs_ref[...]
    qy = ys_ref[...]
    w1t = w1t_ref[...]
    acc = jnp.zeros((MB_C, C3), jnp.float32)
    for k in range(KNN):
        gk = g_ref[0, k]
        f3 = gk[:, 0:C3]
        ux = gk[:, C3:C3 + 1]
        uy = gk[:, C3 + 1:C3 + 2]
        ox = ux - qx
        oy = uy - qy
        on = jnp.sqrt(ox * ox + oy * oy)
        h = (ox * w1t[0:1, :] + oy * w1t[1:2, :] + on * w1t[2:3, :]
             + b1_ref[...])
        h = jnp.where(h >= 0, h, 0.1 * h)
        s = lax.dot_general(h, w2_ref[...], (((1,), (1,)), ((), ())),
                            preferred_element_type=jnp.float32) + b2_ref[...]
        s = 1.0 / (1.0 + jnp.exp(-s))
        acc = acc + s * f3
    o = lax.dot_general(wo_ref[...], acc, (((1,), (1,)), ((), ())),
                        preferred_element_type=jnp.float32) + bo_ref[...]
    out_ref[0] = jnp.where(o >= 0, o, 0.1 * o)


def _mlp_call(g4, xs, ys, w1t, b1r, w2, b2r, wo, bor):
    zmap = lambda b, m: (0, 0)
    return pl.pallas_call(
        _mlp_body,
        grid=(BS, M // MB_C),
        in_specs=[
            pl.BlockSpec((1, KNN, MB_C, DPAD), lambda b, m: (b, 0, m, 0)),
            pl.BlockSpec((MB_C, 1), lambda b, m: (m, 0)),
            pl.BlockSpec((MB_C, 1), lambda b, m: (m, 0)),
            pl.BlockSpec((KNN, 16), zmap),
            pl.BlockSpec((1, 16), zmap),
            pl.BlockSpec((C3, 16), zmap),
            pl.BlockSpec((1, C3), zmap),
            pl.BlockSpec((C3, C3), zmap),
            pl.BlockSpec((C3, 1), zmap),
        ],
        out_specs=pl.BlockSpec((1, C3, MB_C), lambda b, m: (b, 0, m)),
        out_shape=jax.ShapeDtypeStruct((BS, C3, M), jnp.float32),
    )(g4, xs, ys, w1t, b1r, w2, b2r, wo, bor)


def kernel(uv, feat_2d, feat_3d, w1, b1, w2, b2, wo, bo):
    m = jnp.arange(M, dtype=jnp.int32)
    xs = (m % W).astype(jnp.float32).reshape(M, 1)
    ys = (m // W).astype(jnp.float32).reshape(M, 1)

    idx = _knn_call(uv, xs, ys)

    cat = jnp.concatenate([feat_3d, uv], axis=1)
    tab = jnp.transpose(cat, (0, 2, 1))
    tab = jnp.pad(tab, ((0, 0), (0, 0), (0, DPAD - 66)))
    tab = tab.reshape(BS * NP, DPAD)

    idx2 = idx.transpose(0, 2, 1).reshape(BS * KNN * M // WIN, WIN)
    g = _gather_call(idx2, tab)
    g4 = g.reshape(BS, KNN, M, DPAD)

    outc = _mlp_call(
        g4, xs, ys,
        jnp.transpose(w1, (1, 0)), b1.reshape(1, 16),
        w2, b2.reshape(1, C3), wo, bo.reshape(C3, 1))
    return outc.reshape(BS, C3, H, W)


# speedup vs baseline: 64.8919x; 2.1055x over previous
import functools

import jax
import jax.numpy as jnp
from jax import lax
from jax.experimental import pallas as pl
from jax.experimental.pallas import tpu as pltpu
from jax.experimental.pallas import tpu_sc as plsc

BS, H, W = 2, 96, 320
M = H * W
NP = 8192
KNN = 3
C3 = 64
MB_A = 256
MB_C = 512
DPAD = 128
WIN = 120
NWORK = 32
NSTEP = (BS * KNN * M) // (NWORK * WIN)
NBUF = 2


def _topk_body(uv_ref, xs_ref, ys_ref, idx_ref):
    px = uv_ref[0, 0:1, :]
    py = uv_ref[0, 1:2, :]
    qx = xs_ref[...]
    qy = ys_ref[...]
    qq = qx * qx + qy * qy
    pp = px * px + py * py
    qxb = qx.astype(jnp.bfloat16).astype(jnp.float32)
    qyb = qy.astype(jnp.bfloat16).astype(jnp.float32)
    pxb = px.astype(jnp.bfloat16).astype(jnp.float32)
    pyb = py.astype(jnp.bfloat16).astype(jnp.float32)
    qp = qxb * pxb + qyb * pyb
    d = (qq - 2.0 * qp) + pp
    iota = lax.broadcasted_iota(jnp.int32, d.shape, 1)
    inf = jnp.float32(jnp.inf)
    big = jnp.int32(NP)

    def sel(dcur):
        m = jnp.min(dcur, axis=1, keepdims=True)
        e = dcur == m
        i = jnp.min(jnp.where(e, iota, big), axis=1, keepdims=True)
        return i

    i1 = sel(d)
    d2 = jnp.where(iota == i1, inf, d)
    i2 = sel(d2)
    d3 = jnp.where(iota == i2, inf, d2)
    i3 = sel(d3)
    idx_ref[0] = jnp.concatenate([i1, i2, i3], axis=1)


def _knn_call(uv, xs, ys):
    return pl.pallas_call(
        _topk_body,
        grid=(BS, M // MB_A),
        in_specs=[
            pl.BlockSpec((1, 2, NP), lambda b, m: (b, 0, 0)),
            pl.BlockSpec((MB_A, 1), lambda b, m: (m, 0)),
            pl.BlockSpec((MB_A, 1), lambda b, m: (m, 0)),
        ],
        out_specs=pl.BlockSpec((1, MB_A, KNN), lambda b, m: (b, m, 0)),
        out_shape=jax.ShapeDtypeStruct((BS, M, KNN), jnp.int32),
    )(uv, xs, ys)


def _gather_call(idx2, table):
    mesh = plsc.VectorSubcoreMesh(core_axis_name="c", subcore_axis_name="s")
    wpb = KNN * M // WIN

    @functools.partial(
        pl.kernel,
        out_type=jax.ShapeDtypeStruct((BS * KNN * M, DPAD), jnp.float32),
        mesh=mesh,
        scratch_types=(
            [pltpu.VMEM_SHARED((NP, DPAD), jnp.float32),
             pltpu.VMEM((NSTEP, WIN), jnp.int32)]
            + [pltpu.VMEM((WIN, DPAD), jnp.float32) for _ in range(NBUF)]
            + [pltpu.SemaphoreType.DMA for _ in range(NBUF)]
        ),
    )
    def gk(idx_hbm, table_hbm, out_hbm, spm, idx_v, rv0, rv1,
           sm0, sm1):
        bufs = (rv0, rv1)
        sems = (sm0, sm1)
        cid = lax.axis_index("c")
        sid = lax.axis_index("s")
        rows = NP // 16
        pltpu.sync_copy(table_hbm.at[pl.ds(cid * NP + sid * rows, rows)],
                        spm.at[pl.ds(sid * rows, rows)])
        w0 = cid * wpb + sid * NSTEP
        pltpu.sync_copy(idx_hbm.at[pl.ds(w0, NSTEP)], idx_v)
        plsc.subcore_barrier()

        def fire(s, t):
            pltpu.async_copy(spm.at[idx_v.at[s]], bufs[t], sems[t])

        def wait_scatter(s, t):
            pltpu.make_async_copy(
                spm.at[idx_v.at[s]], bufs[t], sems[t]).wait()
            pltpu.sync_copy(
                bufs[t], out_hbm.at[pl.ds((w0 + s) * WIN, WIN)])

        for t in range(NBUF):
            fire(t, t)

        @pl.loop(0, NSTEP - NBUF, step=NBUF)
        def _(g):
            for t in range(NBUF):
                wait_scatter(g + t, t)
                fire(g + t + NBUF, t)

        for t in range(NBUF):
            wait_scatter(NSTEP - NBUF + t, t)

    return gk(idx2, table)


def _mlp_body(g_ref, xs_ref, ys_ref, w1t_ref, b1_ref, w2_ref, b2_ref,
              wo_ref, bo_ref, out_ref):
    qx = xs_ref[...]
    qy = ys_ref[...]
    w1t = w1t_ref[...]
    acc = jnp.zeros((MB_C, C3), jnp.float32)
    for k in range(KNN):
        gk = g_ref[0, k]
        f3 = gk[:, 0:C3]
        ux = gk[:, C3:C3 + 1]
        uy = gk[:, C3 + 1:C3 + 2]
        ox = ux - qx
        oy = uy - qy
        on = jnp.sqrt(ox * ox + oy * oy)
        h = (ox * w1t[0:1, :] + oy * w1t[1:2, :] + on * w1t[2:3, :]
             + b1_ref[...])
        h = jnp.where(h >= 0, h, 0.1 * h)
        s = lax.dot_general(h, w2_ref[...], (((1,), (1,)), ((), ())),
                            preferred_element_type=jnp.float32) + b2_ref[...]
        s = 1.0 / (1.0 + jnp.exp(-s))
        acc = acc + s * f3
    o = lax.dot_general(wo_ref[...], acc, (((1,), (1,)), ((), ())),
                        preferred_element_type=jnp.float32) + bo_ref[...]
    out_ref[0] = jnp.where(o >= 0, o, 0.1 * o)


def _mlp_call(g4, xs, ys, w1t, b1r, w2, b2r, wo, bor):
    zmap = lambda b, m: (0, 0)
    return pl.pallas_call(
        _mlp_body,
        grid=(BS, M // MB_C),
        in_specs=[
            pl.BlockSpec((1, KNN, MB_C, DPAD), lambda b, m: (b, 0, m, 0)),
            pl.BlockSpec((MB_C, 1), lambda b, m: (m, 0)),
            pl.BlockSpec((MB_C, 1), lambda b, m: (m, 0)),
            pl.BlockSpec((KNN, 16), zmap),
            pl.BlockSpec((1, 16), zmap),
            pl.BlockSpec((C3, 16), zmap),
            pl.BlockSpec((1, C3), zmap),
            pl.BlockSpec((C3, C3), zmap),
            pl.BlockSpec((C3, 1), zmap),
        ],
        out_specs=pl.BlockSpec((1, C3, MB_C), lambda b, m: (b, 0, m)),
        out_shape=jax.ShapeDtypeStruct((BS, C3, M), jnp.float32),
    )(g4, xs, ys, w1t, b1r, w2, b2r, wo, bor)


def kernel(uv, feat_2d, feat_3d, w1, b1, w2, b2, wo, bo):
    m = jnp.arange(M, dtype=jnp.int32)
    xs = (m % W).astype(jnp.float32).reshape(M, 1)
    ys = (m // W).astype(jnp.float32).reshape(M, 1)

    idx = _knn_call(uv, xs, ys)

    cat = jnp.concatenate([feat_3d, uv], axis=1)
    tab = jnp.transpose(cat, (0, 2, 1))
    tab = jnp.pad(tab, ((0, 0), (0, 0), (0, DPAD - 66)))
    tab = tab.reshape(BS * NP, DPAD)

    idx2 = idx.transpose(0, 2, 1).reshape(BS * KNN * M // WIN, WIN)
    g = _gather_call(idx2, tab)
    g4 = g.reshape(BS, KNN, M, DPAD)

    outc = _mlp_call(
        g4, xs, ys,
        jnp.transpose(w1, (1, 0)), b1.reshape(1, 16),
        w2, b2.reshape(1, C3), wo, bo.reshape(C3, 1))
    return outc.reshape(BS, C3, H, W)


# speedup vs baseline: 67.8044x; 1.0449x over previous
import functools

import jax
import jax.numpy as jnp
from jax import lax
from jax.experimental import pallas as pl
from jax.experimental.pallas import tpu as pltpu
from jax.experimental.pallas import tpu_sc as plsc

BS, H, W = 2, 96, 320
M = H * W
NP = 8192
KNN = 3
C3 = 64
MB_A = 256
MB_C = 512
DPAD = 128
WIN = 120
NWORK = 32
NSTEP = KNN * M // (NWORK * WIN)
NBUF = 2


def _topk_body(uv_ref, xs_ref, ys_ref, idx_ref):
    px = uv_ref[0:1, :]
    py = uv_ref[1:2, :]
    qx = xs_ref[...]
    qy = ys_ref[...]
    qq = qx * qx + qy * qy
    pp = px * px + py * py
    qm2 = (jnp.concatenate([qx, qy], axis=1) * 2.0).astype(jnp.bfloat16)
    pm = uv_ref[...].astype(jnp.bfloat16)
    qp2 = lax.dot_general(qm2, pm, (((1,), (0,)), ((), ())),
                          preferred_element_type=jnp.float32)
    d = (qq - qp2) + pp
    iota = lax.broadcasted_iota(jnp.int32, d.shape, 1)
    inf = jnp.float32(jnp.inf)
    big = jnp.int32(NP)

    def argmin(d0):
        n = NP // 2
        i0 = lax.broadcasted_iota(jnp.int32, (MB_A, n), 1)
        c = d0[:, n:] < d0[:, :n]
        v = jnp.where(c, d0[:, n:], d0[:, :n])
        ix = jnp.where(c, i0 + n, i0)
        n //= 2
        while n >= 128:
            c = v[:, n:] < v[:, :n]
            v = jnp.where(c, v[:, n:], v[:, :n])
            ix = jnp.where(c, ix[:, n:], ix[:, :n])
            n //= 2
        m = jnp.min(v, axis=1, keepdims=True)
        e = v == m
        return jnp.min(jnp.where(e, ix, big), axis=1, keepdims=True)

    i1 = argmin(d)
    d2 = jnp.where(iota == i1, inf, d)
    i2 = argmin(d2)
    d3 = jnp.where(iota == i2, inf, d2)
    i3 = argmin(d3)
    idx_ref[...] = jnp.concatenate([i1, i2, i3], axis=1)


def _knn_call(uv_b, xs, ys):
    return pl.pallas_call(
        _topk_body,
        grid=(M // MB_A,),
        in_specs=[
            pl.BlockSpec((2, NP), lambda m: (0, 0)),
            pl.BlockSpec((MB_A, 1), lambda m: (m, 0)),
            pl.BlockSpec((MB_A, 1), lambda m: (m, 0)),
        ],
        out_specs=pl.BlockSpec((MB_A, KNN), lambda m: (m, 0)),
        out_shape=jax.ShapeDtypeStruct((M, KNN), jnp.int32),
    )(uv_b, xs, ys)


def _gather_call(idx2, table):
    mesh = plsc.VectorSubcoreMesh(core_axis_name="c", subcore_axis_name="s")

    @functools.partial(
        pl.kernel,
        out_type=jax.ShapeDtypeStruct((KNN * M, DPAD), jnp.float32),
        mesh=mesh,
        scratch_types=(
            [pltpu.VMEM_SHARED((NP, DPAD), jnp.float32),
             pltpu.VMEM((NSTEP, WIN), jnp.int32)]
            + [pltpu.VMEM((WIN, DPAD), jnp.float32) for _ in range(NBUF)]
            + [pltpu.SemaphoreType.DMA for _ in range(NBUF)]
        ),
    )
    def gk(idx_hbm, table_hbm, out_hbm, spm, idx_v, rv0, rv1, sm0, sm1):
        bufs = (rv0, rv1)
        sems = (sm0, sm1)
        cid = lax.axis_index("c")
        sid = lax.axis_index("s")
        wid = sid * 2 + cid
        rows = NP // 16
        pltpu.sync_copy(table_hbm.at[pl.ds(sid * rows, rows)],
                        spm.at[pl.ds(sid * rows, rows)])
        w0 = wid * NSTEP
        pltpu.sync_copy(idx_hbm.at[pl.ds(w0, NSTEP)], idx_v)
        plsc.subcore_barrier()

        def fire(s, t):
            pltpu.async_copy(spm.at[idx_v.at[s]], bufs[t], sems[t])

        def wait_scatter(s, t):
            pltpu.make_async_copy(
                spm.at[idx_v.at[s]], bufs[t], sems[t]).wait()
            pltpu.sync_copy(
                bufs[t], out_hbm.at[pl.ds((w0 + s) * WIN, WIN)])

        for t in range(NBUF):
            fire(t, t)

        @pl.loop(0, NSTEP - NBUF, step=NBUF)
        def _(g):
            for t in range(NBUF):
                wait_scatter(g + t, t)
                fire(g + t + NBUF, t)

        for t in range(NBUF):
            wait_scatter(NSTEP - NBUF + t, t)

    return gk(idx2, table)


def _mlp_body(g_ref, xs_ref, ys_ref, w1t_ref, b1_ref, w2_ref, b2_ref,
              wo_ref, bo_ref, out_ref):
    qx = xs_ref[...]
    qy = ys_ref[...]
    w1t = w1t_ref[...]
    acc = jnp.zeros((MB_C, C3), jnp.float32)
    for k in range(KNN):
        gk = g_ref[k]
        f3 = gk[:, 0:C3]
        ux = gk[:, C3:C3 + 1]
        uy = gk[:, C3 + 1:C3 + 2]
        ox = ux - qx
        oy = uy - qy
        on = jnp.sqrt(ox * ox + oy * oy)
        h = (ox * w1t[0:1, :] + oy * w1t[1:2, :] + on * w1t[2:3, :]
             + b1_ref[...])
        h = jnp.where(h >= 0, h, 0.1 * h)
        s = lax.dot_general(h, w2_ref[...], (((1,), (1,)), ((), ())),
                            preferred_element_type=jnp.float32) + b2_ref[...]
        s = 1.0 / (1.0 + jnp.exp(-s))
        acc = acc + s * f3
    o = lax.dot_general(wo_ref[...], acc, (((1,), (1,)), ((), ())),
                        preferred_element_type=jnp.float32) + bo_ref[...]
    out_ref[...] = jnp.where(o >= 0, o, 0.1 * o)


def _mlp_call(g3, xs, ys, w1t, b1r, w2, b2r, wo, bor):
    zmap = lambda m: (0, 0)
    return pl.pallas_call(
        _mlp_body,
        grid=(M // MB_C,),
        in_specs=[
            pl.BlockSpec((KNN, MB_C, DPAD), lambda m: (0, m, 0)),
            pl.BlockSpec((MB_C, 1), lambda m: (m, 0)),
            pl.BlockSpec((MB_C, 1), lambda m: (m, 0)),
            pl.BlockSpec((KNN, 16), zmap),
            pl.BlockSpec((1, 16), zmap),
            pl.BlockSpec((C3, 16), zmap),
            pl.BlockSpec((1, C3), zmap),
            pl.BlockSpec((C3, C3), zmap),
            pl.BlockSpec((C3, 1), zmap),
        ],
        out_specs=pl.BlockSpec((C3, MB_C), lambda m: (0, m)),
        out_shape=jax.ShapeDtypeStruct((C3, M), jnp.float32),
    )(g3, xs, ys, w1t, b1r, w2, b2r, wo, bor)


def kernel(uv, feat_2d, feat_3d, w1, b1, w2, b2, wo, bo):
    m = jnp.arange(M, dtype=jnp.int32)
    xs = (m % W).astype(jnp.float32).reshape(M, 1)
    ys = (m // W).astype(jnp.float32).reshape(M, 1)

    cat = jnp.concatenate([feat_3d, uv], axis=1)
    tab = jnp.transpose(cat, (0, 2, 1))
    tab = jnp.pad(tab, ((0, 0), (0, 0), (0, DPAD - 66)))

    w1t = jnp.transpose(w1, (1, 0))
    b1r = b1.reshape(1, 16)
    b2r = b2.reshape(1, C3)
    bor = bo.reshape(C3, 1)

    outs = []
    for b in range(BS):
        idx = _knn_call(uv[b], xs, ys)
        idx2 = idx.transpose(1, 0).reshape(KNN * M // WIN, WIN)
        g = _gather_call(idx2, tab[b])
        g3 = g.reshape(KNN, M, DPAD)
        outs.append(_mlp_call(g3, xs, ys, w1t, b1r, w2, b2r, wo, bor))
    return jnp.stack(outs).reshape(BS, C3, H, W)


# speedup vs baseline: 68.2062x; 1.0059x over previous
import functools

import jax
import jax.numpy as jnp
from jax import lax
from jax.experimental import pallas as pl
from jax.experimental.pallas import tpu as pltpu
from jax.experimental.pallas import tpu_sc as plsc

BS, H, W = 2, 96, 320
M = H * W
NP = 8192
KNN = 3
C3 = 64
MB_A = 512
MB_C = 512
DPAD = 128
WIN = 120
NWORK = 32
NSTEP = KNN * M // (NWORK * WIN)
NBUF = 2


def _topk_body(uv_ref, xs_ref, ys_ref, idx_ref):
    px = uv_ref[0:1, :]
    py = uv_ref[1:2, :]
    qx = xs_ref[...]
    qy = ys_ref[...]
    qq = qx * qx + qy * qy
    pp = px * px + py * py
    qm2 = (jnp.concatenate([qx, qy], axis=1) * 2.0).astype(jnp.bfloat16)
    pm = uv_ref[...].astype(jnp.bfloat16)
    qp2 = lax.dot_general(qm2, pm, (((1,), (0,)), ((), ())),
                          preferred_element_type=jnp.float32)
    d = (qq - qp2) + pp
    iota = lax.broadcasted_iota(jnp.int32, d.shape, 1)
    inf = jnp.float32(jnp.inf)
    big = jnp.int32(NP)

    def argmin(d0):
        n = NP // 2
        i0 = lax.broadcasted_iota(jnp.int32, (MB_A, n), 1)
        c = d0[:, n:] < d0[:, :n]
        v = jnp.where(c, d0[:, n:], d0[:, :n])
        ix = jnp.where(c, i0 + n, i0)
        n //= 2
        while n >= 128:
            c = v[:, n:] < v[:, :n]
            v = jnp.where(c, v[:, n:], v[:, :n])
            ix = jnp.where(c, ix[:, n:], ix[:, :n])
            n //= 2
        m = jnp.min(v, axis=1, keepdims=True)
        e = v == m
        return jnp.min(jnp.where(e, ix, big), axis=1, keepdims=True)

    i1 = argmin(d)
    d2 = jnp.where(iota == i1, inf, d)
    i2 = argmin(d2)
    d3 = jnp.where(iota == i2, inf, d2)
    i3 = argmin(d3)
    idx_ref[...] = jnp.concatenate([i1, i2, i3], axis=1)


def _knn_call(uv_b, xs, ys):
    return pl.pallas_call(
        _topk_body,
        grid=(M // MB_A,),
        in_specs=[
            pl.BlockSpec((2, NP), lambda m: (0, 0)),
            pl.BlockSpec((MB_A, 1), lambda m: (m, 0)),
            pl.BlockSpec((MB_A, 1), lambda m: (m, 0)),
        ],
        out_specs=pl.BlockSpec((MB_A, KNN), lambda m: (m, 0)),
        out_shape=jax.ShapeDtypeStruct((M, KNN), jnp.int32),
    )(uv_b, xs, ys)


def _gather_call(idx2, table):
    mesh = plsc.VectorSubcoreMesh(core_axis_name="c", subcore_axis_name="s")

    @functools.partial(
        pl.kernel,
        out_type=jax.ShapeDtypeStruct((KNN * M, DPAD), jnp.float32),
        mesh=mesh,
        scratch_types=(
            [pltpu.VMEM_SHARED((NP, DPAD), jnp.float32),
             pltpu.VMEM((NSTEP, WIN), jnp.int32)]
            + [pltpu.VMEM((WIN, DPAD), jnp.float32) for _ in range(NBUF)]
            + [pltpu.SemaphoreType.DMA for _ in range(NBUF)]
        ),
    )
    def gk(idx_hbm, table_hbm, out_hbm, spm, idx_v, rv0, rv1, sm0, sm1):
        bufs = (rv0, rv1)
        sems = (sm0, sm1)
        cid = lax.axis_index("c")
        sid = lax.axis_index("s")
        wid = sid * 2 + cid
        rows = NP // 16
        pltpu.sync_copy(table_hbm.at[pl.ds(sid * rows, rows)],
                        spm.at[pl.ds(sid * rows, rows)])
        w0 = wid * NSTEP
        pltpu.sync_copy(idx_hbm.at[pl.ds(w0, NSTEP)], idx_v)
        plsc.subcore_barrier()

        def fire(s, t):
            pltpu.async_copy(spm.at[idx_v.at[s]], bufs[t], sems[t])

        def wait_scatter(s, t):
            pltpu.make_async_copy(
                spm.at[idx_v.at[s]], bufs[t], sems[t]).wait()
            pltpu.sync_copy(
                bufs[t], out_hbm.at[pl.ds((w0 + s) * WIN, WIN)])

        for t in range(NBUF):
            fire(t, t)

        @pl.loop(0, NSTEP - NBUF, step=NBUF)
        def _(g):
            for t in range(NBUF):
                wait_scatter(g + t, t)
                fire(g + t + NBUF, t)

        for t in range(NBUF):
            wait_scatter(NSTEP - NBUF + t, t)

    return gk(idx2, table)


def _mlp_body(g_ref, xs_ref, ys_ref, w1t_ref, b1_ref, w2_ref, b2_ref,
              wo_ref, bo_ref, out_ref):
    qx = xs_ref[...]
    qy = ys_ref[...]
    w1t = w1t_ref[...]
    acc = jnp.zeros((MB_C, C3), jnp.float32)
    for k in range(KNN):
        gk = g_ref[k]
        f3 = gk[:, 0:C3]
        ux = gk[:, C3:C3 + 1]
        uy = gk[:, C3 + 1:C3 + 2]
        ox = ux - qx
        oy = uy - qy
        on = jnp.sqrt(ox * ox + oy * oy)
        h = (ox * w1t[0:1, :] + oy * w1t[1:2, :] + on * w1t[2:3, :]
             + b1_ref[...])
        h = jnp.where(h >= 0, h, 0.1 * h)
        s = lax.dot_general(h, w2_ref[...], (((1,), (1,)), ((), ())),
                            preferred_element_type=jnp.float32) + b2_ref[...]
        s = 1.0 / (1.0 + jnp.exp(-s))
        acc = acc + s * f3
    o = lax.dot_general(wo_ref[...], acc, (((1,), (1,)), ((), ())),
                        preferred_element_type=jnp.float32) + bo_ref[...]
    out_ref[...] = jnp.where(o >= 0, o, 0.1 * o)


def _mlp_call(g3, xs, ys, w1t, b1r, w2, b2r, wo, bor):
    zmap = lambda m: (0, 0)
    return pl.pallas_call(
        _mlp_body,
        grid=(M // MB_C,),
        in_specs=[
            pl.BlockSpec((KNN, MB_C, DPAD), lambda m: (0, m, 0)),
            pl.BlockSpec((MB_C, 1), lambda m: (m, 0)),
            pl.BlockSpec((MB_C, 1), lambda m: (m, 0)),
            pl.BlockSpec((KNN, 16), zmap),
            pl.BlockSpec((1, 16), zmap),
            pl.BlockSpec((C3, 16), zmap),
            pl.BlockSpec((1, C3), zmap),
            pl.BlockSpec((C3, C3), zmap),
            pl.BlockSpec((C3, 1), zmap),
        ],
        out_specs=pl.BlockSpec((C3, MB_C), lambda m: (0, m)),
        out_shape=jax.ShapeDtypeStruct((C3, M), jnp.float32),
    )(g3, xs, ys, w1t, b1r, w2, b2r, wo, bor)


def kernel(uv, feat_2d, feat_3d, w1, b1, w2, b2, wo, bo):
    m = jnp.arange(M, dtype=jnp.int32)
    xs = (m % W).astype(jnp.float32).reshape(M, 1)
    ys = (m // W).astype(jnp.float32).reshape(M, 1)

    cat = jnp.concatenate([feat_3d, uv], axis=1)
    tab = jnp.transpose(cat, (0, 2, 1))
    tab = jnp.pad(tab, ((0, 0), (0, 0), (0, DPAD - 66)))

    w1t = jnp.transpose(w1, (1, 0))
    b1r = b1.reshape(1, 16)
    b2r = b2.reshape(1, C3)
    bor = bo.reshape(C3, 1)

    outs = []
    for b in range(BS):
        idx = _knn_call(uv[b], xs, ys)
        idx2 = idx.transpose(1, 0).reshape(KNN * M // WIN, WIN)
        g = _gather_call(idx2, tab[b])
        g3 = g.reshape(KNN, M, DPAD)
        outs.append(_mlp_call(g3, xs, ys, w1t, b1r, w2, b2r, wo, bor))
    return jnp.stack(outs).reshape(BS, C3, H, W)


# speedup vs baseline: 85.2177x; 1.2494x over previous
import functools

import jax
import jax.numpy as jnp
from jax import lax
from jax.experimental import pallas as pl
from jax.experimental.pallas import tpu as pltpu
from jax.experimental.pallas import tpu_sc as plsc

BS, H, W = 2, 96, 320
M = H * W
NP = 8192
KNN = 3
C3 = 64
MB_A = 512
MB_C = 512
DPAD = 128
WIN = 120
NWORK = 32
NSTEP = KNN * M // (NWORK * WIN)
NBUF = 2


def _topk_body(uv_ref, xs_ref, ys_ref, idx_ref):
    px = uv_ref[0:1, :]
    py = uv_ref[1:2, :]
    qx = xs_ref[...]
    qy = ys_ref[...]
    qq = qx * qx + qy * qy
    pp = px * px + py * py
    qm2 = (jnp.concatenate([qx, qy], axis=1) * 2.0).astype(jnp.bfloat16)
    pm = uv_ref[...].astype(jnp.bfloat16)
    qp2 = lax.dot_general(qm2, pm, (((1,), (0,)), ((), ())),
                          preferred_element_type=jnp.float32)
    d = (qq - qp2) + pp
    iotaf = lax.broadcasted_iota(jnp.int32, d.shape, 1).astype(jnp.float32)
    inf = jnp.float32(jnp.inf)
    bigf = jnp.float32(NP)

    def sel(dcur):
        m = jnp.min(dcur, axis=1, keepdims=True)
        return jnp.min(jnp.where(dcur == m, iotaf, bigf), axis=1,
                       keepdims=True)

    i1 = sel(d)
    d2 = jnp.where(iotaf == i1, inf, d)
    i2 = sel(d2)
    d3 = jnp.where(iotaf == i2, inf, d2)
    i3 = sel(d3)
    idx_ref[...] = jnp.concatenate([i1, i2, i3], axis=1).astype(jnp.int32)


def _knn_call(uv_b, xs, ys):
    return pl.pallas_call(
        _topk_body,
        grid=(M // MB_A,),
        in_specs=[
            pl.BlockSpec((2, NP), lambda m: (0, 0)),
            pl.BlockSpec((MB_A, 1), lambda m: (m, 0)),
            pl.BlockSpec((MB_A, 1), lambda m: (m, 0)),
        ],
        out_specs=pl.BlockSpec((MB_A, KNN), lambda m: (m, 0)),
        out_shape=jax.ShapeDtypeStruct((M, KNN), jnp.int32),
    )(uv_b, xs, ys)


def _gather_call(idx2, table):
    mesh = plsc.VectorSubcoreMesh(core_axis_name="c", subcore_axis_name="s")

    @functools.partial(
        pl.kernel,
        out_type=jax.ShapeDtypeStruct((KNN * M, DPAD), jnp.float32),
        mesh=mesh,
        scratch_types=(
            [pltpu.VMEM_SHARED((NP, DPAD), jnp.float32),
             pltpu.VMEM((NSTEP, WIN), jnp.int32)]
            + [pltpu.VMEM((WIN, DPAD), jnp.float32) for _ in range(NBUF)]
            + [pltpu.SemaphoreType.DMA for _ in range(NBUF)]
        ),
    )
    def gk(idx_hbm, table_hbm, out_hbm, spm, idx_v, rv0, rv1, sm0, sm1):
        bufs = (rv0, rv1)
        sems = (sm0, sm1)
        cid = lax.axis_index("c")
        sid = lax.axis_index("s")
        wid = sid * 2 + cid
        rows = NP // 16
        pltpu.sync_copy(table_hbm.at[pl.ds(sid * rows, rows)],
                        spm.at[pl.ds(sid * rows, rows)])
        w0 = wid * NSTEP
        pltpu.sync_copy(idx_hbm.at[pl.ds(w0, NSTEP)], idx_v)
        plsc.subcore_barrier()

        def fire(s, t):
            pltpu.async_copy(spm.at[idx_v.at[s]], bufs[t], sems[t])

        def wait_scatter(s, t):
            pltpu.make_async_copy(
                spm.at[idx_v.at[s]], bufs[t], sems[t]).wait()
            pltpu.sync_copy(
                bufs[t], out_hbm.at[pl.ds((w0 + s) * WIN, WIN)])

        for t in range(NBUF):
            fire(t, t)

        @pl.loop(0, NSTEP - NBUF, step=NBUF)
        def _(g):
            for t in range(NBUF):
                wait_scatter(g + t, t)
                fire(g + t + NBUF, t)

        for t in range(NBUF):
            wait_scatter(NSTEP - NBUF + t, t)

    return gk(idx2, table)


def _mlp_body(g_ref, xs_ref, ys_ref, w1t_ref, b1_ref, w2_ref, b2_ref,
              wo_ref, bo_ref, out_ref):
    qx = xs_ref[...]
    qy = ys_ref[...]
    w1t = w1t_ref[...]
    acc = jnp.zeros((MB_C, C3), jnp.float32)
    for k in range(KNN):
        gk = g_ref[k]
        f3 = gk[:, 0:C3]
        ux = gk[:, C3:C3 + 1]
        uy = gk[:, C3 + 1:C3 + 2]
        ox = ux - qx
        oy = uy - qy
        on = jnp.sqrt(ox * ox + oy * oy)
        h = (ox * w1t[0:1, :] + oy * w1t[1:2, :] + on * w1t[2:3, :]
             + b1_ref[...])
        h = jnp.where(h >= 0, h, 0.1 * h)
        s = lax.dot_general(h, w2_ref[...], (((1,), (1,)), ((), ())),
                            preferred_element_type=jnp.float32) + b2_ref[...]
        s = 1.0 / (1.0 + jnp.exp(-s))
        acc = acc + s * f3
    o = lax.dot_general(wo_ref[...], acc, (((1,), (1,)), ((), ())),
                        preferred_element_type=jnp.float32) + bo_ref[...]
    out_ref[...] = jnp.where(o >= 0, o, 0.1 * o)


def _mlp_call(g3, xs, ys, w1t, b1r, w2, b2r, wo, bor):
    zmap = lambda m: (0, 0)
    return pl.pallas_call(
        _mlp_body,
        grid=(M // MB_C,),
        in_specs=[
            pl.BlockSpec((KNN, MB_C, DPAD), lambda m: (0, m, 0)),
            pl.BlockSpec((MB_C, 1), lambda m: (m, 0)),
            pl.BlockSpec((MB_C, 1), lambda m: (m, 0)),
            pl.BlockSpec((KNN, 16), zmap),
            pl.BlockSpec((1, 16), zmap),
            pl.BlockSpec((C3, 16), zmap),
            pl.BlockSpec((1, C3), zmap),
            pl.BlockSpec((C3, C3), zmap),
            pl.BlockSpec((C3, 1), zmap),
        ],
        out_specs=pl.BlockSpec((C3, MB_C), lambda m: (0, m)),
        out_shape=jax.ShapeDtypeStruct((C3, M), jnp.float32),
    )(g3, xs, ys, w1t, b1r, w2, b2r, wo, bor)


def kernel(uv, feat_2d, feat_3d, w1, b1, w2, b2, wo, bo):
    m = jnp.arange(M, dtype=jnp.int32)
    xs = (m % W).astype(jnp.float32).reshape(M, 1)
    ys = (m // W).astype(jnp.float32).reshape(M, 1)

    cat = jnp.concatenate([feat_3d, uv], axis=1)
    tab = jnp.transpose(cat, (0, 2, 1))
    tab = jnp.pad(tab, ((0, 0), (0, 0), (0, DPAD - 66)))

    w1t = jnp.transpose(w1, (1, 0))
    b1r = b1.reshape(1, 16)
    b2r = b2.reshape(1, C3)
    bor = bo.reshape(C3, 1)

    outs = []
    for b in range(BS):
        idx = _knn_call(uv[b], xs, ys)
        idx2 = idx.transpose(1, 0).reshape(KNN * M // WIN, WIN)
        g = _gather_call(idx2, tab[b])
        g3 = g.reshape(KNN, M, DPAD)
        outs.append(_mlp_call(g3, xs, ys, w1t, b1r, w2, b2r, wo, bor))
    return jnp.stack(outs).reshape(BS, C3, H, W)


# speedup vs baseline: 86.5137x; 1.0152x over previous
import functools

import jax
import jax.numpy as jnp
from jax import lax
from jax.experimental import pallas as pl
from jax.experimental.pallas import tpu as pltpu
from jax.experimental.pallas import tpu_sc as plsc

BS, H, W = 2, 96, 320
M = H * W
NP = 8192
KNN = 3
C3 = 64
MB_A = 512
MB_C = 1024
DPAD = 128
WIN = 120
NWORK = 32
NSTEP = KNN * M // (NWORK * WIN)
NBUF = 2


def _topk_body(uv_ref, xs_ref, ys_ref, idx_ref):
    px = uv_ref[0:1, :]
    py = uv_ref[1:2, :]
    qx = xs_ref[...]
    qy = ys_ref[...]
    qq = qx * qx + qy * qy
    pp = px * px + py * py
    qm2 = (jnp.concatenate([qx, qy], axis=1) * 2.0).astype(jnp.bfloat16)
    pm = uv_ref[...].astype(jnp.bfloat16)
    qp2 = lax.dot_general(qm2, pm, (((1,), (0,)), ((), ())),
                          preferred_element_type=jnp.float32)
    d = (qq - qp2) + pp
    iotaf = lax.broadcasted_iota(jnp.int32, d.shape, 1).astype(jnp.float32)
    inf = jnp.float32(jnp.inf)
    bigf = jnp.float32(NP)

    def sel(dcur):
        m = jnp.min(dcur, axis=1, keepdims=True)
        return jnp.min(jnp.where(dcur == m, iotaf, bigf), axis=1,
                       keepdims=True)

    i1 = sel(d)
    d2 = jnp.where(iotaf == i1, inf, d)
    i2 = sel(d2)
    d3 = jnp.where(iotaf == i2, inf, d2)
    i3 = sel(d3)
    idx_ref[...] = jnp.concatenate([i1, i2, i3], axis=1).astype(jnp.int32)


def _knn_call(uv_b, xs, ys):
    return pl.pallas_call(
        _topk_body,
        grid=(M // MB_A,),
        in_specs=[
            pl.BlockSpec((2, NP), lambda m: (0, 0)),
            pl.BlockSpec((MB_A, 1), lambda m: (m, 0)),
            pl.BlockSpec((MB_A, 1), lambda m: (m, 0)),
        ],
        out_specs=pl.BlockSpec((MB_A, KNN), lambda m: (m, 0)),
        out_shape=jax.ShapeDtypeStruct((M, KNN), jnp.int32),
    )(uv_b, xs, ys)


def _gather_call(idx2, table):
    mesh = plsc.VectorSubcoreMesh(core_axis_name="c", subcore_axis_name="s")

    @functools.partial(
        pl.kernel,
        out_type=jax.ShapeDtypeStruct((KNN * M, DPAD), jnp.float32),
        mesh=mesh,
        scratch_types=(
            [pltpu.VMEM_SHARED((NP, DPAD), jnp.float32),
             pltpu.VMEM((NSTEP, WIN), jnp.int32)]
            + [pltpu.VMEM((WIN, DPAD), jnp.float32) for _ in range(NBUF)]
            + [pltpu.SemaphoreType.DMA for _ in range(NBUF)]
        ),
    )
    def gk(idx_hbm, table_hbm, out_hbm, spm, idx_v, rv0, rv1, sm0, sm1):
        bufs = (rv0, rv1)
        sems = (sm0, sm1)
        cid = lax.axis_index("c")
        sid = lax.axis_index("s")
        wid = sid * 2 + cid
        rows = NP // 16
        pltpu.sync_copy(table_hbm.at[pl.ds(sid * rows, rows)],
                        spm.at[pl.ds(sid * rows, rows)])
        w0 = wid * NSTEP
        pltpu.sync_copy(idx_hbm.at[pl.ds(w0, NSTEP)], idx_v)
        plsc.subcore_barrier()

        def fire(s, t):
            pltpu.async_copy(spm.at[idx_v.at[s]], bufs[t], sems[t])

        def wait_scatter(s, t):
            pltpu.make_async_copy(
                spm.at[idx_v.at[s]], bufs[t], sems[t]).wait()
            pltpu.sync_copy(
                bufs[t], out_hbm.at[pl.ds((w0 + s) * WIN, WIN)])

        for t in range(NBUF):
            fire(t, t)

        @pl.loop(0, NSTEP - NBUF, step=NBUF)
        def _(g):
            for t in range(NBUF):
                wait_scatter(g + t, t)
                fire(g + t + NBUF, t)

        for t in range(NBUF):
            wait_scatter(NSTEP - NBUF + t, t)

    return gk(idx2, table)


def _mlp_body(g_ref, xs_ref, ys_ref, w1t_ref, b1_ref, w2_ref, b2_ref,
              wo_ref, bo_ref, out_ref):
    qx = xs_ref[...]
    qy = ys_ref[...]
    w1t = w1t_ref[...]
    acc = jnp.zeros((MB_C, C3), jnp.float32)
    for k in range(KNN):
        gk = g_ref[k]
        f3 = gk[:, 0:C3]
        ux = gk[:, C3:C3 + 1]
        uy = gk[:, C3 + 1:C3 + 2]
        ox = ux - qx
        oy = uy - qy
        on = jnp.sqrt(ox * ox + oy * oy)
        h = (ox * w1t[0:1, :] + oy * w1t[1:2, :] + on * w1t[2:3, :]
             + b1_ref[...])
        h = jnp.where(h >= 0, h, 0.1 * h)
        s = lax.dot_general(h, w2_ref[...], (((1,), (1,)), ((), ())),
                            preferred_element_type=jnp.float32) + b2_ref[...]
        s = 1.0 / (1.0 + jnp.exp(-s))
        acc = acc + s * f3
    o = lax.dot_general(wo_ref[...], acc, (((1,), (1,)), ((), ())),
                        preferred_element_type=jnp.float32) + bo_ref[...]
    out_ref[...] = jnp.where(o >= 0, o, 0.1 * o)


def _mlp_call(g3, xs, ys, w1t, b1r, w2, b2r, wo, bor):
    zmap = lambda m: (0, 0)
    return pl.pallas_call(
        _mlp_body,
        grid=(M // MB_C,),
        in_specs=[
            pl.BlockSpec((KNN, MB_C, DPAD), lambda m: (0, m, 0)),
            pl.BlockSpec((MB_C, 1), lambda m: (m, 0)),
            pl.BlockSpec((MB_C, 1), lambda m: (m, 0)),
            pl.BlockSpec((KNN, 16), zmap),
            pl.BlockSpec((1, 16), zmap),
            pl.BlockSpec((C3, 16), zmap),
            pl.BlockSpec((1, C3), zmap),
            pl.BlockSpec((C3, C3), zmap),
            pl.BlockSpec((C3, 1), zmap),
        ],
        out_specs=pl.BlockSpec((C3, MB_C), lambda m: (0, m)),
        out_shape=jax.ShapeDtypeStruct((C3, M), jnp.float32),
    )(g3, xs, ys, w1t, b1r, w2, b2r, wo, bor)


def kernel(uv, feat_2d, feat_3d, w1, b1, w2, b2, wo, bo):
    m = jnp.arange(M, dtype=jnp.int32)
    xs = (m % W).astype(jnp.float32).reshape(M, 1)
    ys = (m // W).astype(jnp.float32).reshape(M, 1)

    cat = jnp.concatenate([feat_3d, uv], axis=1)
    tab = jnp.transpose(cat, (0, 2, 1))
    tab = jnp.pad(tab, ((0, 0), (0, 0), (0, DPAD - 66)))

    w1t = jnp.transpose(w1, (1, 0))
    b1r = b1.reshape(1, 16)
    b2r = b2.reshape(1, C3)
    bor = bo.reshape(C3, 1)

    outs = []
    for b in range(BS):
        idx = _knn_call(uv[b], xs, ys)
        idx2 = idx.transpose(1, 0).reshape(KNN * M // WIN, WIN)
        g = _gather_call(idx2, tab[b])
        g3 = g.reshape(KNN, M, DPAD)
        outs.append(_mlp_call(g3, xs, ys, w1t, b1r, w2, b2r, wo, bor))
    return jnp.stack(outs).reshape(BS, C3, H, W)
